# baseline (device time: 79513 ns/iter reference)
import jax
import jax.numpy as jnp
from jax import lax
from jax.experimental import pallas as pl
from jax.experimental.pallas import tpu as pltpu

N_DEV = 4
B_LOC = 2
SQ = 512
SKV = 512
H_PER = 8
DH = 64
HD = H_PER * DH
D_MODEL = 768
WINDOW = 128

ABLATE_COMM = False


def kernel(x, Wq, K_ext, V_ext, Wo):
    my = lax.axis_index("i")

    x16 = x.astype(jnp.bfloat16)
    wq16 = Wq.astype(jnp.bfloat16)
    wo16 = Wo.astype(jnp.bfloat16)

    k_my = lax.dynamic_slice_in_dim(K_ext, my * B_LOC, B_LOC, axis=0)
    v_my = lax.dynamic_slice_in_dim(V_ext, my * B_LOC, B_LOC, axis=0)
    k16 = k_my.astype(jnp.bfloat16).reshape(B_LOC, SKV, N_DEV * HD)
    v16 = v_my.astype(jnp.bfloat16).reshape(B_LOC, SKV, N_DEV * HD)

    def body(x_ref, wq_ref, k_ref, v_ref, wo_ref, out_ref,
             wqg, wog, send_sems, recv_sems):
        my_i = lax.axis_index("i")

        barrier = pltpu.get_barrier_semaphore()
        for d in range(1, N_DEV):
            pl.semaphore_signal(
                barrier, inc=1,
                device_id=((my_i + d) % N_DEV,),
                device_id_type=pl.DeviceIdType.MESH,
            )
        pl.semaphore_wait(barrier, N_DEV - 1)

        if ABLATE_COMM:
            for j in range(N_DEV):
                wqg[j] = wq_ref[...]
                wog[j] = wo_ref[...]
            sends = []
        else:
            wqg[my_i] = wq_ref[...]
            wog[my_i] = wo_ref[...]

            sends = []
            for t, src, gat in ((0, wq_ref, wqg), (1, wo_ref, wog)):
                for d in range(1, N_DEV):
                    rdma = pltpu.make_async_remote_copy(
                        src_ref=src,
                        dst_ref=gat.at[my_i],
                        send_sem=send_sems.at[t, d - 1],
                        recv_sem=recv_sems.at[t, d - 1],
                        device_id=((my_i + d) % N_DEV,),
                        device_id_type=pl.DeviceIdType.MESH,
                    )
                    rdma.start()
                    sends.append(rdma)

        def wait_recvs(t, gat):
            if ABLATE_COMM:
                return
            for d in range(1, N_DEV):
                recv = pltpu.make_async_remote_copy(
                    src_ref=gat.at[0],
                    dst_ref=gat.at[0],
                    send_sem=send_sems.at[t, d - 1],
                    recv_sem=recv_sems.at[t, d - 1],
                    device_id=(my_i,),
                    device_id_type=pl.DeviceIdType.MESH,
                )
                recv.wait_recv()

        qi = lax.broadcasted_iota(jnp.int32, (SQ, SKV), 0)
        ki = lax.broadcasted_iota(jnp.int32, (SQ, SKV), 1)
        maskadd = jnp.where(jnp.abs(qi - ki) <= WINDOW,
                            jnp.float32(0.0), jnp.float32(-1e9))

        wait_recvs(0, wqg)

        ctx_blocks = []
        for b in range(B_LOC):
            xb = x_ref[b]
            for j in range(N_DEV):
                qj = lax.dot_general(
                    xb, wqg[j], (((1,), (0,)), ((), ())),
                    preferred_element_type=jnp.float32)
                qj = (qj * 0.125).astype(jnp.bfloat16)
                ctx_parts = []
                for h in range(H_PER):
                    lo = j * HD + h * DH
                    qh = qj[:, h * DH:(h + 1) * DH]
                    kh = k_ref[b][:, lo:lo + DH]
                    s = lax.dot_general(
                        qh, kh, (((1,), (1,)), ((), ())),
                        preferred_element_type=jnp.float32)
                    w = jnp.exp(s + maskadd)
                    denom = jnp.sum(w, axis=1, keepdims=True)
                    ctxh = lax.dot_general(
                        w.astype(jnp.bfloat16), v_ref[b][:, lo:lo + DH],
                        (((1,), (0,)), ((), ())),
                        preferred_element_type=jnp.float32)
                    ctx_parts.append((ctxh / denom).astype(jnp.bfloat16))
                ctx_blocks.append(jnp.concatenate(ctx_parts, axis=1))

        wait_recvs(1, wog)

        for b in range(B_LOC):
            acc = jnp.zeros((SQ, D_MODEL), jnp.float32)
            for j in range(N_DEV):
                acc = acc + lax.dot_general(
                    ctx_blocks[b * N_DEV + j], wog[j], (((1,), (0,)), ((), ())),
                    preferred_element_type=jnp.float32)
            out_ref[b] = acc

        for rdma in sends:
            rdma.wait_send()

    return pl.pallas_call(
        body,
        out_shape=jax.ShapeDtypeStruct((B_LOC, SQ, D_MODEL), jnp.float32),
        in_specs=[pl.BlockSpec(memory_space=pltpu.VMEM)] * 5,
        out_specs=pl.BlockSpec(memory_space=pltpu.VMEM),
        scratch_shapes=[
            pltpu.VMEM((N_DEV, D_MODEL, HD), jnp.bfloat16),
            pltpu.VMEM((N_DEV, HD, D_MODEL), jnp.bfloat16),
            pltpu.SemaphoreType.DMA((2, N_DEV - 1)),
            pltpu.SemaphoreType.DMA((2, N_DEV - 1)),
        ],
        compiler_params=pltpu.CompilerParams(
            collective_id=0,
            vmem_limit_bytes=100 * 1024 * 1024,
        ),
    )(x16, wq16, k16, v16, wo16)


# device time: 77082 ns/iter; 1.0315x vs baseline; 1.0315x over previous
import jax
import jax.numpy as jnp
from jax import lax
from jax.experimental import pallas as pl
from jax.experimental.pallas import tpu as pltpu

N_DEV = 4
B_LOC = 2
SQ = 512
SKV = 512
H_PER = 8
DH = 64
HD = H_PER * DH
D_MODEL = 768
WINDOW = 128

ABLATE_COMM = False


def kernel(x, Wq, K_ext, V_ext, Wo):
    my = lax.axis_index("i")

    x16 = x.astype(jnp.bfloat16)
    wq16 = (Wq * 0.125).astype(jnp.bfloat16)
    wo16 = Wo.astype(jnp.bfloat16)

    k_my = lax.dynamic_slice_in_dim(K_ext, my * B_LOC, B_LOC, axis=0)
    v_my = lax.dynamic_slice_in_dim(V_ext, my * B_LOC, B_LOC, axis=0)
    k_t = k_my.astype(jnp.bfloat16).reshape(B_LOC, SKV, N_DEV * HD)
    k_t = k_t.transpose(0, 2, 1)
    v16 = v_my.astype(jnp.bfloat16).reshape(B_LOC, SKV, N_DEV * HD)

    def body(x_ref, wq_ref, k_ref, v_ref, wo_ref, out_ref,
             wqg, wog, send_sems, recv_sems):
        my_i = lax.axis_index("i")

        barrier = pltpu.get_barrier_semaphore()
        for d in range(1, N_DEV):
            pl.semaphore_signal(
                barrier, inc=1,
                device_id=((my_i + d) % N_DEV,),
                device_id_type=pl.DeviceIdType.MESH,
            )
        pl.semaphore_wait(barrier, N_DEV - 1)

        if ABLATE_COMM:
            for j in range(N_DEV):
                wqg[j] = wq_ref[...]
                wog[j] = wo_ref[...]
            sends = []
        else:
            wqg[my_i] = wq_ref[...]
            wog[my_i] = wo_ref[...]

            sends = []
            for t, src, gat in ((0, wq_ref, wqg), (1, wo_ref, wog)):
                for d in range(1, N_DEV):
                    rdma = pltpu.make_async_remote_copy(
                        src_ref=src,
                        dst_ref=gat.at[my_i],
                        send_sem=send_sems.at[t, d - 1],
                        recv_sem=recv_sems.at[t, d - 1],
                        device_id=((my_i + d) % N_DEV,),
                        device_id_type=pl.DeviceIdType.MESH,
                    )
                    rdma.start()
                    sends.append(rdma)

        def wait_recvs(t, gat):
            if ABLATE_COMM:
                return
            for d in range(1, N_DEV):
                recv = pltpu.make_async_remote_copy(
                    src_ref=gat.at[0],
                    dst_ref=gat.at[0],
                    send_sem=send_sems.at[t, d - 1],
                    recv_sem=recv_sems.at[t, d - 1],
                    device_id=(my_i,),
                    device_id_type=pl.DeviceIdType.MESH,
                )
                recv.wait_recv()

        qi = lax.broadcasted_iota(jnp.int32, (SQ, SKV), 0)
        ki = lax.broadcasted_iota(jnp.int32, (SQ, SKV), 1)
        maskadd = jnp.where(jnp.abs(qi - ki) <= WINDOW,
                            jnp.float32(0.0), jnp.float32(-1e9))

        wait_recvs(0, wqg)

        ctx_blocks = []
        for b in range(B_LOC):
            xb = x_ref[b]
            for j in range(N_DEV):
                qj = lax.dot_general(
                    xb, wqg[j], (((1,), (0,)), ((), ())),
                    preferred_element_type=jnp.float32).astype(jnp.bfloat16)
                ctx_parts = []
                for h in range(H_PER):
                    lo = j * HD + h * DH
                    qh = qj[:, h * DH:(h + 1) * DH]
                    kh = k_ref[b][lo:lo + DH, :]
                    s = lax.dot_general(
                        qh, kh, (((1,), (0,)), ((), ())),
                        preferred_element_type=jnp.float32)
                    w = jnp.exp((s + maskadd).astype(jnp.bfloat16))
                    denom = jnp.sum(w, axis=1, keepdims=True,
                                    dtype=jnp.float32)
                    ctxh = lax.dot_general(
                        w, v_ref[b][:, lo:lo + DH],
                        (((1,), (0,)), ((), ())),
                        preferred_element_type=jnp.float32)
                    ctx_parts.append((ctxh / denom).astype(jnp.bfloat16))
                ctx_blocks.append(jnp.concatenate(ctx_parts, axis=1))

        wait_recvs(1, wog)

        for b in range(B_LOC):
            acc = jnp.zeros((SQ, D_MODEL), jnp.float32)
            for j in range(N_DEV):
                acc = acc + lax.dot_general(
                    ctx_blocks[b * N_DEV + j], wog[j], (((1,), (0,)), ((), ())),
                    preferred_element_type=jnp.float32)
            out_ref[b] = acc

        for rdma in sends:
            rdma.wait_send()

    return pl.pallas_call(
        body,
        out_shape=jax.ShapeDtypeStruct((B_LOC, SQ, D_MODEL), jnp.float32),
        in_specs=[pl.BlockSpec(memory_space=pltpu.VMEM)] * 5,
        out_specs=pl.BlockSpec(memory_space=pltpu.VMEM),
        scratch_shapes=[
            pltpu.VMEM((N_DEV, D_MODEL, HD), jnp.bfloat16),
            pltpu.VMEM((N_DEV, HD, D_MODEL), jnp.bfloat16),
            pltpu.SemaphoreType.DMA((2, N_DEV - 1)),
            pltpu.SemaphoreType.DMA((2, N_DEV - 1)),
        ],
        compiler_params=pltpu.CompilerParams(
            collective_id=0,
            vmem_limit_bytes=100 * 1024 * 1024,
        ),
    )(x16, wq16, k_t, v16, wo16)


# device time: 63325 ns/iter; 1.2556x vs baseline; 1.2172x over previous
import jax
import jax.numpy as jnp
from jax import lax
from jax.experimental import pallas as pl
from jax.experimental.pallas import tpu as pltpu

N_DEV = 4
B_LOC = 2
SQ = 512
SKV = 512
H_PER = 8
DH = 64
HD = H_PER * DH
D_MODEL = 768
WINDOW = 128


def kernel(x, Wq, K_ext, V_ext, Wo):
    my = lax.axis_index("i")

    x16 = x.astype(jnp.bfloat16)
    wq16 = Wq.astype(jnp.bfloat16)
    wo16 = Wo.astype(jnp.bfloat16)

    k_my = lax.dynamic_slice_in_dim(K_ext, my * B_LOC, B_LOC, axis=0)
    v_my = lax.dynamic_slice_in_dim(V_ext, my * B_LOC, B_LOC, axis=0)
    k_t = k_my.astype(jnp.bfloat16).reshape(B_LOC, SKV, N_DEV * HD)
    k_t = k_t.transpose(0, 2, 1)
    v16 = v_my.astype(jnp.bfloat16).reshape(B_LOC, SKV, N_DEV * HD)

    def body(x_ref, wq_ref, k_ref, v_ref, wo_ref, out_ref,
             wqg, wog, send_sems, recv_sems):
        my_i = lax.axis_index("i")

        barrier = pltpu.get_barrier_semaphore()
        for d in range(1, N_DEV):
            pl.semaphore_signal(
                barrier, inc=1,
                device_id=((my_i + d) % N_DEV,),
                device_id_type=pl.DeviceIdType.MESH,
            )
        pl.semaphore_wait(barrier, N_DEV - 1)

        wqg[my_i] = wq_ref[...]
        wog[my_i] = wo_ref[...]

        sends = []
        for t, src, gat in ((0, wq_ref, wqg), (1, wo_ref, wog)):
            for d in range(1, N_DEV):
                rdma = pltpu.make_async_remote_copy(
                    src_ref=src,
                    dst_ref=gat.at[my_i],
                    send_sem=send_sems.at[t, d - 1],
                    recv_sem=recv_sems.at[t, d - 1],
                    device_id=((my_i + d) % N_DEV,),
                    device_id_type=pl.DeviceIdType.MESH,
                )
                rdma.start()
                sends.append(rdma)

        def wait_recvs(t, gat):
            for d in range(1, N_DEV):
                recv = pltpu.make_async_remote_copy(
                    src_ref=gat.at[0],
                    dst_ref=gat.at[0],
                    send_sem=send_sems.at[t, d - 1],
                    recv_sem=recv_sems.at[t, d - 1],
                    device_id=(my_i,),
                    device_id_type=pl.DeviceIdType.MESH,
                )
                recv.wait_recv()

        qi = lax.broadcasted_iota(jnp.int32, (SQ, SKV), 0)
        ki = lax.broadcasted_iota(jnp.int32, (SQ, SKV), 1)
        maskadd = jnp.where(jnp.abs(qi - ki) <= WINDOW,
                            jnp.float32(0.0), jnp.float32(-1e9))

        wait_recvs(0, wqg)

        ctx_blocks = []
        for b in range(B_LOC):
            xb = x_ref[b]
            for j in range(N_DEV):
                qj = lax.dot_general(
                    xb, wqg[j], (((1,), (0,)), ((), ())),
                    preferred_element_type=jnp.float32)
                qj = (qj * 0.125).astype(jnp.bfloat16)
                ctx_parts = []
                for h in range(H_PER):
                    lo = j * HD + h * DH
                    qh = qj[:, h * DH:(h + 1) * DH]
                    kh = k_ref[b][lo:lo + DH, :]
                    s = lax.dot_general(
                        qh, kh, (((1,), (0,)), ((), ())),
                        preferred_element_type=jnp.float32)
                    w = jnp.exp(s + maskadd)
                    denom = jnp.sum(w, axis=1, keepdims=True)
                    ctxh = lax.dot_general(
                        w.astype(jnp.bfloat16), v_ref[b][:, lo:lo + DH],
                        (((1,), (0,)), ((), ())),
                        preferred_element_type=jnp.float32)
                    ctx_parts.append((ctxh / denom).astype(jnp.bfloat16))
                ctx_blocks.append(jnp.concatenate(ctx_parts, axis=1))

        wait_recvs(1, wog)

        for b in range(B_LOC):
            acc = jnp.zeros((SQ, D_MODEL), jnp.float32)
            for j in range(N_DEV):
                acc = acc + lax.dot_general(
                    ctx_blocks[b * N_DEV + j], wog[j], (((1,), (0,)), ((), ())),
                    preferred_element_type=jnp.float32)
            out_ref[b] = acc

        for rdma in sends:
            rdma.wait_send()

    return pl.pallas_call(
        body,
        out_shape=jax.ShapeDtypeStruct((B_LOC, SQ, D_MODEL), jnp.float32),
        in_specs=[pl.BlockSpec(memory_space=pltpu.VMEM)] * 5,
        out_specs=pl.BlockSpec(memory_space=pltpu.VMEM),
        scratch_shapes=[
            pltpu.VMEM((N_DEV, D_MODEL, HD), jnp.bfloat16),
            pltpu.VMEM((N_DEV, HD, D_MODEL), jnp.bfloat16),
            pltpu.SemaphoreType.DMA((2, N_DEV - 1)),
            pltpu.SemaphoreType.DMA((2, N_DEV - 1)),
        ],
        compiler_params=pltpu.CompilerParams(collective_id=0),
    )(x16, wq16, k_t, v16, wo16)
